# baseline (speedup 1.0000x reference)
import functools

import jax
import jax.numpy as jnp
from jax import lax
from jax.experimental import pallas as pl
from jax.experimental.pallas import tpu as pltpu
from jax.experimental.pallas import tpu_sc as plsc

NEG = 0.2
HEADS = 4
NC, NS, L = 2, 16, 16
EDGE_TYPES = [('outfit', 'wears', 'item'), ('item', 'worn_by', 'outfit'),
              ('outfit', 'similar', 'outfit'), ('outfit', 'same_age', 'outfit'),
              ('item', 'cooccurs', 'item')]
HOMO = {('outfit', 'similar', 'outfit'), ('outfit', 'same_age', 'outfit'),
        ('item', 'cooccurs', 'item')}


def _cdiv(a, b):
    return (a + b - 1) // b


def _mm_kern(x_ref, w_ref, b_ref, o_ref):
    o_ref[...] = (jnp.dot(x_ref[...], w_ref[...],
                          preferred_element_type=jnp.float32) + b_ref[...])


def _mm_ln_elu_kern(x_ref, w_ref, b_ref, g_ref, bb_ref, o_ref):
    y = (jnp.dot(x_ref[...], w_ref[...],
                 preferred_element_type=jnp.float32) + b_ref[...])
    m = y.mean(-1, keepdims=True)
    v = ((y - m) ** 2).mean(-1, keepdims=True)
    y = (y - m) / jnp.sqrt(v + 1e-5) * g_ref[...] + bb_ref[...]
    o_ref[...] = jnp.where(y > 0, y, jnp.exp(y) - 1.0)


def _tc_linear(x, w, b, ln_g=None, ln_b=None, block_m=1024):
    n, k = x.shape
    o = w.shape[0]
    pad = (-n) % block_m
    xp = jnp.pad(x, ((0, pad), (0, 0))) if pad else x
    np_ = xp.shape[0]
    args = [xp, w.T, b[None]]
    if ln_g is not None:
        kern = _mm_ln_elu_kern
        args += [ln_g[None], ln_b[None]]
    else:
        kern = _mm_kern
    extra = [pl.BlockSpec((1, o), lambda i: (0, 0))] * (len(args) - 2)
    out = pl.pallas_call(
        kern,
        grid=(np_ // block_m,),
        in_specs=[pl.BlockSpec((block_m, k), lambda i: (i, 0)),
                  pl.BlockSpec((k, o), lambda i: (0, 0))] + extra,
        out_specs=pl.BlockSpec((block_m, o), lambda i: (i, 0)),
        out_shape=jax.ShapeDtypeStruct((np_, o), jnp.float32),
    )(*args)
    return out[:n] if pad else out


def _make_comb_kern(nm):
    def kern(*refs):
        msgs = refs[:nm]
        x_ref, bm_ref, g_ref, bb_ref, o_ref = refs[nm:]
        agg = msgs[0][...]
        for mr in msgs[1:]:
            agg = agg + mr[...]
        y = agg * (1.0 / nm) + bm_ref[...] + x_ref[...]
        m = y.mean(-1, keepdims=True)
        v = ((y - m) ** 2).mean(-1, keepdims=True)
        y = (y - m) / jnp.sqrt(v + 1e-5) * g_ref[...] + bb_ref[...]
        o_ref[...] = jnp.where(y > 0, y, jnp.exp(y) - 1.0)
    return kern


def _tc_combine(msgs, x, bias_mean, ln_g, ln_b, block_m=1024):
    n = x.shape[0]
    pad = (-n) % block_m
    msgs = [jnp.pad(m, ((0, pad), (0, 0))) if pad else m for m in msgs]
    xp = jnp.pad(x, ((0, pad), (0, 0))) if pad else x
    np_ = xp.shape[0]
    nm = len(msgs)
    big = pl.BlockSpec((block_m, 128), lambda i: (i, 0))
    one = pl.BlockSpec((1, 128), lambda i: (0, 0))
    out = pl.pallas_call(
        _make_comb_kern(nm),
        grid=(np_ // block_m,),
        in_specs=[big] * nm + [big, one, one, one],
        out_specs=big,
        out_shape=jax.ShapeDtypeStruct((np_, 128), jnp.float32),
    )(*msgs, xp, bias_mean[None], ln_g[None], ln_b[None])
    return out[:n] if pad else out


@functools.partial(jax.jit, static_argnames=('g_', 'ch', 'n_chunks', 'et', 'concat'))
def _sc_edge(xlcat, xrcat, src, dst, att, tok, *, g_, ch, n_chunks, et, concat):
    G = g_
    f = G * 128
    HG = HEADS // G
    C = 128 // HG
    CHP = ch + 8
    R = ch // NS
    FZ = 32 if R % 32 == 0 else 16
    FF = 48 if R % 48 == 0 else FZ
    NBLK = et // L
    W = G * L
    n_src = xlcat.shape[0] // G
    n_dst = xrcat.shape[0] // G
    mesh = plsc.VectorSubcoreMesh(core_axis_name="c", subcore_axis_name="s")

    @functools.partial(
        pl.kernel, mesh=mesh,
        compiler_params=pltpu.CompilerParams(needs_layout_passes=False),
        out_type=jax.ShapeDtypeStruct((n_chunks * ch, 128), jnp.float32),
        scratch_types=(
            [pltpu.VMEM((et,), jnp.int32)] * 2
            + [pltpu.VMEM((et + 2 * L,), jnp.int32)] * 2
            + [pltpu.VMEM((W,), jnp.int32)] * 2
            + [pltpu.VMEM((W, 128), jnp.float32)] * 2
            + [pltpu.VMEM((L, 128), jnp.float32)] * G
            + [pltpu.VMEM((L, 128), jnp.float32)]
            + [pltpu.VMEM((f,), jnp.float32)]
            + [pltpu.VMEM((L,), jnp.float32)]
            + [pltpu.VMEM((FZ, 128), jnp.float32)]
            + [pltpu.VMEM((FF, 128), jnp.float32)] * 3
            + [pltpu.VMEM_SHARED((CHP, 128), jnp.float32)] * G
            + [pltpu.VMEM_SHARED((CHP, 128), jnp.float32)]
            + [pltpu.SemaphoreType.DMA] * 2
        ),
    )
    def k(*refs):
        it = iter(refs)
        (xl_h, xr_h, src_h, dst_h, att_h, tok_h, out_h,
         src_v, dst_v, csrc_v, cloc_v, idxs_v, idxd_v, xj_v, xi_v) = (
            next(it) for _ in range(15))
        nsc_v = [next(it) for _ in range(G)]
        dsc_v, att_v, tok_v, zb_v, fn_v, fd_v, fo_v = (
            next(it) for _ in range(7))
        num_a = [next(it) for _ in range(G)]
        den_a, sem1, sem2 = next(it), next(it), next(it)

        cid = lax.axis_index("c")
        sid = lax.axis_index("s")
        zero16 = jnp.zeros((L,), jnp.float32)

        pltpu.sync_copy(src_h.at[pl.ds(sid * et, et)], src_v)
        pltpu.sync_copy(dst_h.at[pl.ds(sid * et, et)], dst_v)
        pltpu.sync_copy(att_h, att_v)
        pltpu.sync_copy(tok_h, tok_v)

        def zfill(r, _):
            for fb in range(128 // L):
                zb_v[r, pl.ds(fb * L, L)] = zero16
            return 0
        lax.fori_loop(0, FZ, zfill, 0)

        nmy = n_chunks // 2

        def cbody(i, _):
            chunk = i + cid * nmy
            lo = chunk * ch
            hi = lo + ch

            def zbody(j, _):
                r0 = sid * R + j * FZ
                for g in range(G):
                    pltpu.sync_copy(zb_v, num_a[g].at[pl.ds(r0, FZ)])
                pltpu.sync_copy(zb_v, den_a.at[pl.ds(r0, FZ)])
                return 0
            lax.fori_loop(0, R // FZ, zbody, 0)
            plsc.subcore_barrier()

            def pbody(blk, m):
                d0 = dst_v[pl.ds(blk * 2 * L, L)]
                s0 = src_v[pl.ds(blk * 2 * L, L)]
                d1 = dst_v[pl.ds(blk * 2 * L + L, L)]
                s1 = src_v[pl.ds(blk * 2 * L + L, L)]
                m0 = (d0 >= lo) & (d0 < hi)
                m1 = (d1 >= lo) & (d1 < hi)
                i0 = m0.astype(jnp.int32)
                i1 = m1.astype(jnp.int32)
                c0 = plsc.cumsum(i0)
                c1 = plsc.cumsum(i1)
                n0 = c0[L - 1]
                p0 = jnp.where(m0, m + c0 - i0, et + L)
                p1 = jnp.where(m1, m + n0 + c1 - i1, et + L)
                plsc.store_scatter(csrc_v, [p0], s0)
                plsc.store_scatter(cloc_v, [p0], d0 - lo)
                plsc.store_scatter(csrc_v, [p1], s1)
                plsc.store_scatter(cloc_v, [p1], d1 - lo)
                return m + n0 + c1[L - 1]
            m_tot = lax.fori_loop(0, NBLK // 2, pbody, 0)

            def qbody(blk, _):
                base = blk * L
                lanes = lax.broadcasted_iota(jnp.int32, (L,), 0)
                valid = lanes < (m_tot - base)
                sv = jnp.where(valid, csrc_v[pl.ds(base, L)], 0)
                lv_raw = cloc_v[pl.ds(base, L)]
                lv = jnp.where(valid, lv_raw, ch)
                gv = jnp.where(valid, lv_raw + lo, 0)
                for g in range(G):
                    idxs_v[pl.ds(g * L, L)] = sv + g * n_src
                    idxd_v[pl.ds(g * L, L)] = gv + g * n_dst
                cp1 = pltpu.async_copy(xl_h.at[idxs_v], xj_v, sem1)
                cp2 = pltpu.async_copy(xr_h.at[idxd_v], xi_v, sem2)
                cp1.wait()
                cp2.wait()
                iot = lax.broadcasted_iota(jnp.int32, (L,), 0)

                def one_row(r):
                    valid_r = r < (m_tot - base)
                    exvs = []
                    for h in range(HEADS):
                        g, hh = h // HG, h % HG
                        acc = zero16
                        for vblk in range(C // L):
                            fo = hh * C + vblk * L
                            xv = (xj_v[g * L + r, pl.ds(fo, L)]
                                  + xi_v[g * L + r, pl.ds(fo, L)])
                            lr = jnp.where(xv >= 0, xv, xv * NEG)
                            acc = acc + lr * att_v[pl.ds(g * 128 + fo, L)]
                        av = jnp.full((L,), jnp.sum(acc), jnp.float32)
                        ev = jnp.where(valid_r, jnp.exp(av), 0.0)
                        exvs.append(ev)
                    drow = jnp.where(
                        iot == 0, exvs[0], jnp.where(
                            iot == 1, exvs[1], jnp.where(
                                iot == 2, exvs[2], jnp.where(
                                    iot == 3, exvs[3], 0.0))))
                    dsc_v[r, pl.ds(0, L)] = drow
                    for fb in range(1, 128 // L):
                        dsc_v[r, pl.ds(fb * L, L)] = zero16
                    for g in range(G):
                        for fb in range(128 // L):
                            h = g * HG + fb // (C // L)
                            nsc_v[g][r, pl.ds(fb * L, L)] = (
                                xj_v[g * L + r, pl.ds(fb * L, L)] * exvs[h])

                if G == 1:
                    for r in range(L):
                        one_row(r)
                else:
                    def rbody(rr, _):
                        for kk in range(4):
                            one_row(rr * 4 + kk)
                        return 0
                    lax.fori_loop(0, L // 4, rbody, 0)

                for g in range(G):
                    pltpu.sync_copy(nsc_v[g], num_a[g].at[lv], add=True)
                pltpu.sync_copy(dsc_v, den_a.at[lv], add=True)
                return 0
            lax.fori_loop(0, (m_tot + L - 1) // L, qbody, 0)
            plsc.subcore_barrier()

            def fbody(j, _):
                r0 = sid * R + j * FF
                g0 = chunk * ch + r0
                pltpu.sync_copy(den_a.at[pl.ds(r0, FF)], fd_v)
                for g in range(G):
                    pltpu.sync_copy(num_a[g].at[pl.ds(r0, FF)], fn_v)

                    def frow(r, _):
                        rcpv = 1.0 / (fd_v[r, pl.ds(0, L)] + 1e-16)
                        if concat:
                            for fb in range(128 // L):
                                h = fb // (C // L)
                                rc = jnp.full((L,), rcpv[h], jnp.float32)
                                fo_v[r, pl.ds(fb * L, L)] = (
                                    fn_v[r, pl.ds(fb * L, L)] * rc)
                        else:
                            rc = jnp.full((L,), rcpv[g] * (1.0 / HEADS),
                                          jnp.float32)
                            for cb in range(128 // L):
                                val = fn_v[r, pl.ds(cb * L, L)] * rc
                                if g > 0:
                                    val = val + fo_v[r, pl.ds(cb * L, L)]
                                fo_v[r, pl.ds(cb * L, L)] = val
                        return 0
                    lax.fori_loop(0, FF, frow, 0)
                pltpu.sync_copy(fo_v, out_h.at[pl.ds(g0, FF)])
                return 0
            lax.fori_loop(0, R // FF, fbody, 0)
            return 0
        lax.fori_loop(0, nmy, cbody, 0)

    return k(xlcat, xrcat, src, dst, att, tok)


def _edge_arrays(ei, n_dst, homo):
    src, dst = ei[0], ei[1]
    if homo:
        loop = jnp.arange(n_dst, dtype=ei.dtype)
        src = jnp.concatenate([src, loop])
        dst = jnp.concatenate([dst, loop])
    e = src.shape[0]
    epad = _cdiv(e, NS * L) * NS * L
    if epad != e:
        src = jnp.pad(src, (0, epad - e))
        dst = jnp.pad(dst, (0, epad - e), constant_values=-1)
    return src, dst, epad


def _relation_msg(conv_p, xls, xrs, src, dst, epad, last, tok):
    ch = 768 if last else 2560
    n_dst = xrs[0].shape[0]
    n_chunks = 2 * _cdiv(n_dst, 2 * ch)
    att = conv_p['att'].reshape(-1)
    xlcat = xls[0] if len(xls) == 1 else jnp.concatenate(xls, axis=0)
    xrcat = xrs[0] if len(xrs) == 1 else jnp.concatenate(xrs, axis=0)
    msg = _sc_edge(xlcat, xrcat, src, dst, att, tok, g_=len(xls), ch=ch,
                   n_chunks=n_chunks, et=epad // NS, concat=not last)
    return msg[:n_dst]


def kernel(x_outfit, x_item, edges, params):
    po, pi = params['outfit_proj'], params['item_proj']
    x = {'outfit': _tc_linear(x_outfit, po['lin']['w'], po['lin']['b'],
                              po['ln_g'], po['ln_b']),
         'item': _tc_linear(x_item, pi['lin']['w'], pi['lin']['b'],
                            pi['ln_g'], pi['ln_b'])}

    tok = jnp.zeros((16,), jnp.float32)
    n_nodes = {'outfit': x_outfit.shape[0], 'item': x_item.shape[0]}
    eprep = {}
    for (s, r, d) in EDGE_TYPES:
        eprep[r] = _edge_arrays(edges[r], n_nodes[d], (s, r, d) in HOMO)

    for li in range(2):
        last = (li == 1)
        f = 512 if last else 128
        lp = params['layers'][li]

        need = {'outfit': [], 'item': []}
        for (s, r, d) in EDGE_TYPES:
            key = s + '__' + r + '__' + d
            need[s].append((key, 'lin_l'))
            need[d].append((key, 'lin_r'))
        xt = {}
        for nt in ('outfit', 'item'):
            wcat = jnp.concatenate(
                [lp['convs'][k][w]['w'] for (k, w) in need[nt]], axis=0)
            bcat = jnp.concatenate(
                [lp['convs'][k][w]['b'] for (k, w) in need[nt]], axis=0)
            big = _tc_linear(x[nt], wcat, bcat)
            xt[nt] = {}
            for j, (k, w) in enumerate(need[nt]):
                xt[nt][(k, w)] = [big[:, j * f + g * 128:j * f + (g + 1) * 128]
                                  for g in range(f // 128)]

        msgs = {'outfit': [], 'item': []}
        for (s, r, d) in EDGE_TYPES:
            key = s + '__' + r + '__' + d
            src, dst, epad = eprep[r]
            msg = _relation_msg(lp['convs'][key], xt[s][(key, 'lin_l')],
                                xt[d][(key, 'lin_r')], src, dst, epad, last,
                                tok)
            tok = msg[0, :16]
            msgs[d].append(msg)

        newx = {}
        for nt in ('outfit', 'item'):
            biases = [lp['convs'][k]['bias'] for (k, _) in need[nt]
                      if _ == 'lin_r']
            bias_mean = sum(biases) / len(biases)
            newx[nt] = _tc_combine(msgs[nt], x[nt], bias_mean,
                                   lp['ln_g'], lp['ln_b'])
        x = newx

    ep = params['embed_proj']
    return (_tc_linear(x['outfit'], ep['w'], ep['b']),
            _tc_linear(x['item'], ep['w'], ep['b']))


# speedup vs baseline: 1.2035x; 1.2035x over previous
import functools

import jax
import jax.numpy as jnp
from jax import lax
from jax.experimental import pallas as pl
from jax.experimental.pallas import tpu as pltpu
from jax.experimental.pallas import tpu_sc as plsc

NEG = 0.2
HEADS = 4
NC, NS, L = 2, 16, 16
EDGE_TYPES = [('outfit', 'wears', 'item'), ('item', 'worn_by', 'outfit'),
              ('outfit', 'similar', 'outfit'), ('outfit', 'same_age', 'outfit'),
              ('item', 'cooccurs', 'item')]
HOMO = {('outfit', 'similar', 'outfit'), ('outfit', 'same_age', 'outfit'),
        ('item', 'cooccurs', 'item')}


def _cdiv(a, b):
    return (a + b - 1) // b


def _mm_kern(x_ref, w_ref, b_ref, o_ref):
    o_ref[...] = (jnp.dot(x_ref[...], w_ref[...],
                          preferred_element_type=jnp.float32) + b_ref[...])


def _mm_ln_elu_kern(x_ref, w_ref, b_ref, g_ref, bb_ref, o_ref):
    y = (jnp.dot(x_ref[...], w_ref[...],
                 preferred_element_type=jnp.float32) + b_ref[...])
    m = y.mean(-1, keepdims=True)
    v = ((y - m) ** 2).mean(-1, keepdims=True)
    y = (y - m) / jnp.sqrt(v + 1e-5) * g_ref[...] + bb_ref[...]
    o_ref[...] = jnp.where(y > 0, y, jnp.exp(y) - 1.0)


def _tc_linear(x, w, b, ln_g=None, ln_b=None, block_m=1024):
    n, k = x.shape
    o = w.shape[0]
    pad = (-n) % block_m
    xp = jnp.pad(x, ((0, pad), (0, 0))) if pad else x
    np_ = xp.shape[0]
    args = [xp, w.T, b[None]]
    if ln_g is not None:
        kern = _mm_ln_elu_kern
        args += [ln_g[None], ln_b[None]]
    else:
        kern = _mm_kern
    extra = [pl.BlockSpec((1, o), lambda i: (0, 0))] * (len(args) - 2)
    out = pl.pallas_call(
        kern,
        grid=(np_ // block_m,),
        in_specs=[pl.BlockSpec((block_m, k), lambda i: (i, 0)),
                  pl.BlockSpec((k, o), lambda i: (0, 0))] + extra,
        out_specs=pl.BlockSpec((block_m, o), lambda i: (i, 0)),
        out_shape=jax.ShapeDtypeStruct((np_, o), jnp.float32),
    )(*args)
    return out[:n] if pad else out


def _make_comb_kern(nm):
    def kern(*refs):
        msgs = refs[:nm]
        x_ref, bm_ref, g_ref, bb_ref, o_ref = refs[nm:]
        agg = msgs[0][...]
        for mr in msgs[1:]:
            agg = agg + mr[...]
        y = agg * (1.0 / nm) + bm_ref[...] + x_ref[...]
        m = y.mean(-1, keepdims=True)
        v = ((y - m) ** 2).mean(-1, keepdims=True)
        y = (y - m) / jnp.sqrt(v + 1e-5) * g_ref[...] + bb_ref[...]
        o_ref[...] = jnp.where(y > 0, y, jnp.exp(y) - 1.0)
    return kern


def _tc_combine(msgs, x, bias_mean, ln_g, ln_b, block_m=1024):
    n = x.shape[0]
    pad = (-n) % block_m
    msgs = [jnp.pad(m, ((0, pad), (0, 0))) if pad else m for m in msgs]
    xp = jnp.pad(x, ((0, pad), (0, 0))) if pad else x
    np_ = xp.shape[0]
    nm = len(msgs)
    big = pl.BlockSpec((block_m, 128), lambda i: (i, 0))
    one = pl.BlockSpec((1, 128), lambda i: (0, 0))
    out = pl.pallas_call(
        _make_comb_kern(nm),
        grid=(np_ // block_m,),
        in_specs=[big] * nm + [big, one, one, one],
        out_specs=big,
        out_shape=jax.ShapeDtypeStruct((np_, 128), jnp.float32),
    )(*msgs, xp, bias_mean[None], ln_g[None], ln_b[None])
    return out[:n] if pad else out


@functools.partial(jax.jit, static_argnames=('g_', 'ch', 'n_chunks', 'et', 'concat'))
def _sc_edge(xlcat, xrcat, src, dst, att, tok, *, g_, ch, n_chunks, et, concat):
    G = g_
    f = G * 128
    HG = HEADS // G
    C = 128 // HG
    CHP = ch + 8
    R = ch // NS
    FZ = 32 if R % 32 == 0 else 16
    FF = 48 if R % 48 == 0 else FZ
    NBLK = et // L
    W = G * L
    n_src = xlcat.shape[0] // G
    n_dst = xrcat.shape[0] // G
    mesh = plsc.VectorSubcoreMesh(core_axis_name="c", subcore_axis_name="s")

    @functools.partial(
        pl.kernel, mesh=mesh,
        compiler_params=pltpu.CompilerParams(needs_layout_passes=False),
        out_type=jax.ShapeDtypeStruct((n_chunks * ch, 128), jnp.float32),
        scratch_types=(
            [pltpu.VMEM((et,), jnp.int32)] * 2
            + [pltpu.VMEM((et + 2 * L,), jnp.int32)] * 2
            + [pltpu.VMEM((W,), jnp.int32)] * 2
            + [pltpu.VMEM((W, 128), jnp.float32)] * 2
            + [pltpu.VMEM((L, 128), jnp.float32)] * G
            + [pltpu.VMEM((L, 128), jnp.float32)]
            + [pltpu.VMEM((f,), jnp.float32)]
            + [pltpu.VMEM((L,), jnp.float32)]
            + [pltpu.VMEM((FZ, 128), jnp.float32)]
            + [pltpu.VMEM((FF, 128), jnp.float32)] * 3
            + [pltpu.VMEM_SHARED((CHP, 128), jnp.float32)] * G
            + [pltpu.VMEM_SHARED((CHP, 128), jnp.float32)]
            + [pltpu.SemaphoreType.DMA] * 2
        ),
    )
    def k(*refs):
        it = iter(refs)
        (xl_h, xr_h, src_h, dst_h, att_h, tok_h, out_h,
         src_v, dst_v, csrc_v, cloc_v, idxs_v, idxd_v, xj_v, xi_v) = (
            next(it) for _ in range(15))
        nsc_v = [next(it) for _ in range(G)]
        dsc_v, att_v, tok_v, zb_v, fn_v, fd_v, fo_v = (
            next(it) for _ in range(7))
        num_a = [next(it) for _ in range(G)]
        den_a, sem1, sem2 = next(it), next(it), next(it)

        cid = lax.axis_index("c")
        sid = lax.axis_index("s")
        zero16 = jnp.zeros((L,), jnp.float32)

        pltpu.sync_copy(src_h.at[pl.ds(sid * et, et)], src_v)
        pltpu.sync_copy(dst_h.at[pl.ds(sid * et, et)], dst_v)
        pltpu.sync_copy(att_h, att_v)
        pltpu.sync_copy(tok_h, tok_v)

        def zfill(r, _):
            for fb in range(128 // L):
                zb_v[r, pl.ds(fb * L, L)] = zero16
            return 0
        lax.fori_loop(0, FZ, zfill, 0)

        nmy = n_chunks // 2

        def cbody(i, _):
            chunk = i + cid * nmy
            lo = chunk * ch
            hi = lo + ch

            def zbody(j, _):
                r0 = sid * R + j * FZ
                for g in range(G):
                    pltpu.sync_copy(zb_v, num_a[g].at[pl.ds(r0, FZ)])
                pltpu.sync_copy(zb_v, den_a.at[pl.ds(r0, FZ)])
                return 0
            lax.fori_loop(0, R // FZ, zbody, 0)
            plsc.subcore_barrier()

            def pbody(blk, m):
                d0 = dst_v[pl.ds(blk * 2 * L, L)]
                s0 = src_v[pl.ds(blk * 2 * L, L)]
                d1 = dst_v[pl.ds(blk * 2 * L + L, L)]
                s1 = src_v[pl.ds(blk * 2 * L + L, L)]
                m0 = (d0 >= lo) & (d0 < hi)
                m1 = (d1 >= lo) & (d1 < hi)
                i0 = m0.astype(jnp.int32)
                i1 = m1.astype(jnp.int32)
                c0 = plsc.cumsum(i0)
                c1 = plsc.cumsum(i1)
                n0 = c0[L - 1]
                p0 = jnp.where(m0, m + c0 - i0, et + L)
                p1 = jnp.where(m1, m + n0 + c1 - i1, et + L)
                plsc.store_scatter(csrc_v, [p0], s0)
                plsc.store_scatter(cloc_v, [p0], d0 - lo)
                plsc.store_scatter(csrc_v, [p1], s1)
                plsc.store_scatter(cloc_v, [p1], d1 - lo)
                return m + n0 + c1[L - 1]
            m_tot = lax.fori_loop(0, NBLK // 2, pbody, 0)

            def qbody(blk, _):
                base = blk * L
                lanes = lax.broadcasted_iota(jnp.int32, (L,), 0)
                valid = lanes < (m_tot - base)
                sv = jnp.where(valid, csrc_v[pl.ds(base, L)], 0)
                lv_raw = cloc_v[pl.ds(base, L)]
                lv = jnp.where(valid, lv_raw, ch)
                gv = jnp.where(valid, lv_raw + lo, 0)
                for g in range(G):
                    idxs_v[pl.ds(g * L, L)] = sv + g * n_src
                    idxd_v[pl.ds(g * L, L)] = gv + g * n_dst
                cp1 = pltpu.async_copy(xl_h.at[idxs_v], xj_v, sem1)
                cp2 = pltpu.async_copy(xr_h.at[idxd_v], xi_v, sem2)
                cp1.wait()
                cp2.wait()
                iot = lax.broadcasted_iota(jnp.int32, (L,), 0)

                def one_row(r):
                    valid_r = r < (m_tot - base)
                    exvs = []
                    for h in range(HEADS):
                        g, hh = h // HG, h % HG
                        acc = zero16
                        for vblk in range(C // L):
                            fo = hh * C + vblk * L
                            xv = (xj_v[g * L + r, pl.ds(fo, L)]
                                  + xi_v[g * L + r, pl.ds(fo, L)])
                            lr = jnp.where(xv >= 0, xv, xv * NEG)
                            acc = acc + lr * att_v[pl.ds(g * 128 + fo, L)]
                        av = jnp.full((L,), jnp.sum(acc), jnp.float32)
                        ev = jnp.where(valid_r, jnp.exp(av), 0.0)
                        exvs.append(ev)
                    drow = jnp.where(
                        iot == 0, exvs[0], jnp.where(
                            iot == 1, exvs[1], jnp.where(
                                iot == 2, exvs[2], jnp.where(
                                    iot == 3, exvs[3], 0.0))))
                    dsc_v[r, pl.ds(0, L)] = drow
                    for fb in range(1, 128 // L):
                        dsc_v[r, pl.ds(fb * L, L)] = zero16
                    for g in range(G):
                        for fb in range(128 // L):
                            h = g * HG + fb // (C // L)
                            nsc_v[g][r, pl.ds(fb * L, L)] = (
                                xj_v[g * L + r, pl.ds(fb * L, L)] * exvs[h])

                if G == 1:
                    for r in range(L):
                        one_row(r)
                else:
                    def rbody(rr, _):
                        one_row(rr)
                        return 0
                    lax.fori_loop(0, L, rbody, 0)

                for g in range(G):
                    pltpu.sync_copy(nsc_v[g], num_a[g].at[lv], add=True)
                pltpu.sync_copy(dsc_v, den_a.at[lv], add=True)
                return 0
            lax.fori_loop(0, (m_tot + L - 1) // L, qbody, 0)
            plsc.subcore_barrier()

            def fbody(j, _):
                r0 = sid * R + j * FF
                g0 = chunk * ch + r0
                pltpu.sync_copy(den_a.at[pl.ds(r0, FF)], fd_v)
                for g in range(G):
                    pltpu.sync_copy(num_a[g].at[pl.ds(r0, FF)], fn_v)

                    def frow(r, _):
                        rcpv = 1.0 / (fd_v[r, pl.ds(0, L)] + 1e-16)
                        if concat:
                            for fb in range(128 // L):
                                h = fb // (C // L)
                                rc = jnp.full((L,), rcpv[h], jnp.float32)
                                fo_v[r, pl.ds(fb * L, L)] = (
                                    fn_v[r, pl.ds(fb * L, L)] * rc)
                        else:
                            rc = jnp.full((L,), rcpv[g] * (1.0 / HEADS),
                                          jnp.float32)
                            for cb in range(128 // L):
                                val = fn_v[r, pl.ds(cb * L, L)] * rc
                                if g > 0:
                                    val = val + fo_v[r, pl.ds(cb * L, L)]
                                fo_v[r, pl.ds(cb * L, L)] = val
                        return 0
                    lax.fori_loop(0, FF, frow, 0)
                pltpu.sync_copy(fo_v, out_h.at[pl.ds(g0, FF)])
                return 0
            lax.fori_loop(0, R // FF, fbody, 0)
            return 0
        lax.fori_loop(0, nmy, cbody, 0)

    return k(xlcat, xrcat, src, dst, att, tok)


def _edge_arrays(ei, n_dst, homo):
    src, dst = ei[0], ei[1]
    if homo:
        loop = jnp.arange(n_dst, dtype=ei.dtype)
        src = jnp.concatenate([src, loop])
        dst = jnp.concatenate([dst, loop])
    e = src.shape[0]
    epad = _cdiv(e, NS * L) * NS * L
    if epad != e:
        src = jnp.pad(src, (0, epad - e))
        dst = jnp.pad(dst, (0, epad - e), constant_values=-1)
    return src, dst, epad


def _relation_msg(conv_p, xls, xrs, src, dst, epad, last, tok):
    ch = 768 if last else 2560
    n_dst = xrs[0].shape[0]
    n_chunks = 2 * _cdiv(n_dst, 2 * ch)
    att = conv_p['att'].reshape(-1)
    xlcat = xls[0] if len(xls) == 1 else jnp.concatenate(xls, axis=0)
    xrcat = xrs[0] if len(xrs) == 1 else jnp.concatenate(xrs, axis=0)
    msg = _sc_edge(xlcat, xrcat, src, dst, att, tok, g_=len(xls), ch=ch,
                   n_chunks=n_chunks, et=epad // NS, concat=not last)
    return msg[:n_dst]


def kernel(x_outfit, x_item, edges, params):
    po, pi = params['outfit_proj'], params['item_proj']
    x = {'outfit': _tc_linear(x_outfit, po['lin']['w'], po['lin']['b'],
                              po['ln_g'], po['ln_b']),
         'item': _tc_linear(x_item, pi['lin']['w'], pi['lin']['b'],
                            pi['ln_g'], pi['ln_b'])}

    tok = jnp.zeros((16,), jnp.float32)
    n_nodes = {'outfit': x_outfit.shape[0], 'item': x_item.shape[0]}
    eprep = {}
    for (s, r, d) in EDGE_TYPES:
        eprep[r] = _edge_arrays(edges[r], n_nodes[d], (s, r, d) in HOMO)

    for li in range(2):
        last = (li == 1)
        f = 512 if last else 128
        lp = params['layers'][li]

        need = {'outfit': [], 'item': []}
        for (s, r, d) in EDGE_TYPES:
            key = s + '__' + r + '__' + d
            need[s].append((key, 'lin_l'))
            need[d].append((key, 'lin_r'))
        xt = {}
        for nt in ('outfit', 'item'):
            wcat = jnp.concatenate(
                [lp['convs'][k][w]['w'] for (k, w) in need[nt]], axis=0)
            bcat = jnp.concatenate(
                [lp['convs'][k][w]['b'] for (k, w) in need[nt]], axis=0)
            big = _tc_linear(x[nt], wcat, bcat)
            xt[nt] = {}
            for j, (k, w) in enumerate(need[nt]):
                xt[nt][(k, w)] = [big[:, j * f + g * 128:j * f + (g + 1) * 128]
                                  for g in range(f // 128)]

        msgs = {'outfit': [], 'item': []}
        for (s, r, d) in EDGE_TYPES:
            key = s + '__' + r + '__' + d
            src, dst, epad = eprep[r]
            msg = _relation_msg(lp['convs'][key], xt[s][(key, 'lin_l')],
                                xt[d][(key, 'lin_r')], src, dst, epad, last,
                                tok)
            tok = msg[0, :16]
            msgs[d].append(msg)

        newx = {}
        for nt in ('outfit', 'item'):
            biases = [lp['convs'][k]['bias'] for (k, _) in need[nt]
                      if _ == 'lin_r']
            bias_mean = sum(biases) / len(biases)
            newx[nt] = _tc_combine(msgs[nt], x[nt], bias_mean,
                                   lp['ln_g'], lp['ln_b'])
        x = newx

    ep = params['embed_proj']
    return (_tc_linear(x['outfit'], ep['w'], ep['b']),
            _tc_linear(x['item'], ep['w'], ep['b']))
